# baseline (speedup 1.0000x reference)
import functools

import jax
import jax.numpy as jnp
from jax import lax
from jax.experimental import pallas as pl
from jax.experimental.pallas import tpu as pltpu
from jax.experimental.pallas import tpu_sc as plsc

_NC = 2
_NS = 16
_NW = _NC * _NS
_L = 16

_N = 8192
_D = 1024
_ROWS_PER_WORKER = _N // _NW
_R = 32
_CHUNKS = _ROWS_PER_WORKER // _R
_DG = _D // _L


@functools.partial(
    pl.kernel,
    mesh=plsc.VectorSubcoreMesh(core_axis_name="c", subcore_axis_name="s"),
    compiler_params=pltpu.CompilerParams(needs_layout_passes=False),
    out_type=jax.ShapeDtypeStruct((_N * _D,), jnp.float32),
    scratch_types=[
        pltpu.VMEM((_D,), jnp.int32),
        pltpu.VMEM((_R * _D,), jnp.float32),
        pltpu.VMEM((_R * _D,), jnp.float32),
    ],
)
def _permute_sc(in_hbm, perm_hbm, out_hbm, idx_v, in_v, out_v):
    wid = lax.axis_index("s") * _NC + lax.axis_index("c")
    base = wid * _ROWS_PER_WORKER * _D
    pltpu.sync_copy(perm_hbm, idx_v)

    def chunk_body(c, carry):
        off = base + c * (_R * _D)
        pltpu.sync_copy(in_hbm.at[pl.ds(off, _R * _D)], in_v)

        def dg_body(dg, inner):
            idx = idx_v[pl.ds(dg * _L, _L)]
            for r in range(_R):
                vals = plsc.load_gather(in_v, [idx + (r * _D)])
                out_v[pl.ds(dg * _L + r * _D, _L)] = vals
            return inner

        lax.fori_loop(0, _DG, dg_body, 0)
        pltpu.sync_copy(out_v, out_hbm.at[pl.ds(off, _R * _D)])
        return carry

    lax.fori_loop(0, _CHUNKS, chunk_body, 0)


def kernel(target, permutation):
    b, s, d = target.shape
    out = _permute_sc(target.reshape(b * s * d), permutation)
    return out.reshape(b, s, d)


# speedup vs baseline: 1.5476x; 1.5476x over previous
import functools

import jax
import jax.numpy as jnp
from jax import lax
from jax.experimental import pallas as pl
from jax.experimental.pallas import tpu as pltpu
from jax.experimental.pallas import tpu_sc as plsc

_NC = 2
_NS = 16
_NW = _NC * _NS
_L = 16

_N = 8192
_D = 1024
_ROWS_PER_WORKER = _N // _NW
_R = 16
_CHUNKS = _ROWS_PER_WORKER // _R
_CH = _R * _D
_DG = _D // _L


@functools.partial(
    pl.kernel,
    mesh=plsc.VectorSubcoreMesh(core_axis_name="c", subcore_axis_name="s"),
    compiler_params=pltpu.CompilerParams(needs_layout_passes=False),
    out_type=jax.ShapeDtypeStruct((_N * _D,), jnp.float32),
    scratch_types=[
        pltpu.VMEM((_D,), jnp.int32),
        pltpu.VMEM((_CH,), jnp.float32),
        pltpu.VMEM((_CH,), jnp.float32),
        pltpu.VMEM((_CH,), jnp.float32),
        pltpu.VMEM((_CH,), jnp.float32),
        pltpu.SemaphoreType.DMA,
        pltpu.SemaphoreType.DMA,
        pltpu.SemaphoreType.DMA,
        pltpu.SemaphoreType.DMA,
    ],
)
def _permute_sc(in_hbm, perm_hbm, out_hbm, idx_v, in_v0, in_v1, out_v0,
                out_v1, sem_i0, sem_i1, sem_o0, sem_o1):
    wid = lax.axis_index("s") * _NC + lax.axis_index("c")
    base = wid * _ROWS_PER_WORKER * _D
    pltpu.sync_copy(perm_hbm, idx_v)

    in_vs, out_vs = (in_v0, in_v1), (out_v0, out_v1)
    sem_is, sem_os = (sem_i0, sem_i1), (sem_o0, sem_o1)

    def in_copy(c):
        b = c % 2
        return pltpu.make_async_copy(
            in_hbm.at[pl.ds(base + c * _CH, _CH)], in_vs[b], sem_is[b])

    def out_copy(c):
        b = c % 2
        return pltpu.make_async_copy(
            out_vs[b], out_hbm.at[pl.ds(base + c * _CH, _CH)], sem_os[b])

    def compute(in_b, out_b):
        @plsc.parallel_loop(0, _DG, unroll=2)
        def dg_body(dg):
            idx = idx_v[pl.ds(dg * _L, _L)]
            for r in range(_R):
                vals = plsc.load_gather(in_b, [idx + (r * _D)])
                out_b[pl.ds(dg * _L + r * _D, _L)] = vals

    in_copy(0).start()
    in_copy(1).start()
    for c in range(_CHUNKS):
        b = c % 2
        in_copy(c).wait()
        if c >= 2:
            out_copy(c - 2).wait()
        compute(in_vs[b], out_vs[b])
        out_copy(c).start()
        if c + 2 < _CHUNKS:
            in_copy(c + 2).start()
    out_copy(_CHUNKS - 2).wait()
    out_copy(_CHUNKS - 1).wait()


def kernel(target, permutation):
    b, s, d = target.shape
    out = _permute_sc(target.reshape(b * s * d), permutation)
    return out.reshape(b, s, d)
